# baseline (speedup 1.0000x reference)
import jax
import jax.numpy as jnp
from jax import lax
from jax.experimental import pallas as pl
from jax.experimental.pallas import tpu as pltpu, tpu_sc as plsc

NC, NS, LANES = 2, 16, 16
NW = NC * NS
D = 64
BBLK = 128
NBUF = 4
EPS = 1e-5
GRP = BBLK // LANES


def _rsqrt_v(v):
    i = plsc.bitcast(v, jnp.int32)
    y = plsc.bitcast(jnp.full((LANES,), 0x5F3759DF, jnp.int32) - (i >> 1),
                     jnp.float32)
    for _ in range(3):
        y = y * (1.5 - 0.5 * v * y * y)
    return y


def _body(xT_hbm, table_hbm, w_hbm, b_hbm, out_hbm,
          idxs, gbufs, obufs, w_v, b_v, sis, sos):
    L, B = xT_hbm.shape
    per_w = B // NW
    blocks = per_w // BBLK
    n_iter = L * blocks
    wid = lax.axis_index("s") * NC + lax.axis_index("c")
    bt0 = wid * blocks
    pltpu.sync_copy(w_hbm, w_v)
    pltpu.sync_copy(b_hbm, b_v)
    iota = lax.iota(jnp.int32, LANES)

    def start_in(i, k):
        l = i // blocks
        bt = bt0 + lax.rem(i, blocks)
        pltpu.sync_copy(xT_hbm.at[l, pl.ds(bt * BBLK, BBLK)], idxs[k])
        pltpu.async_copy(table_hbm.at[idxs[k]], gbufs[k], sis[k])

    def wait_in(k):
        pltpu.make_async_copy(table_hbm.at[pl.ds(0, BBLK)], gbufs[k],
                              sis[k]).wait()

    def start_out(i, k):
        l = i // blocks
        bt = bt0 + lax.rem(i, blocks)
        pltpu.async_copy(obufs[k], out_hbm.at[l, :, bt, :], sos[k])

    def wait_out(k):
        pltpu.make_async_copy(obufs[k], out_hbm.at[0, :, 0, :], sos[k]).wait()

    def compute(k):
        gbuf, obuf = gbufs[k], obufs[k]
        means, rstds = [], []
        zero = jnp.zeros((LANES,), jnp.float32)
        for g in range(GRP):
            rows = iota + g * LANES

            def d_sum(d, carry):
                s, s2 = carry
                v = plsc.load_gather(gbuf, [rows, jnp.full((LANES,), d,
                                                           jnp.int32)])
                return s + v, s2 + v * v

            s, s2 = lax.fori_loop(0, D, d_sum, (zero, zero), unroll=4)
            mean = s * (1.0 / D)
            var = s2 * (1.0 / D) - mean * mean
            means.append(mean)
            rstds.append(_rsqrt_v(var + EPS))

        rows_g = [iota + g * LANES for g in range(GRP)]

        def d_norm(d, _):
            dv = jnp.full((LANES,), d, jnp.int32)
            wsp = plsc.load_gather(w_v, [dv])
            bsp = plsc.load_gather(b_v, [dv])
            dt = d >> 3
            dcol = (d & 7) * BBLK
            for g in range(GRP):
                v = plsc.load_gather(gbuf, [rows_g[g], dv])
                o = (v - means[g]) * (rstds[g] * wsp) + bsp
                obuf[dt, pl.ds(dcol + g * LANES, LANES)] = o
            return 0

        lax.fori_loop(0, D, d_norm, 0, unroll=2)

    start_in(0, 0)
    start_in(1, 1)

    def outer(i4, _):
        for k in range(NBUF):
            i = i4 * NBUF + k
            wait_in(k)
            compute(k)
            start_out(i, k)
            kn = (k + 2) % NBUF
            if k >= 2:
                wait_out(kn)
            else:
                @pl.when(i4 > 0)
                def _():
                    wait_out(kn)

            @pl.when(i + 2 < n_iter)
            def _():
                start_in(i + 2, kn)
        return 0

    lax.fori_loop(0, n_iter // NBUF, outer, 0)
    wait_out((n_iter - 2) % NBUF)
    wait_out((n_iter - 1) % NBUF)


def kernel(x, table, ln_weight, ln_bias):
    B, L = x.shape
    NT = B // BBLK

    def body(xT_hbm, table_hbm, w_hbm, b_hbm, out_hbm, *scratch):
        idxs = scratch[0:NBUF]
        gbufs = scratch[NBUF:2 * NBUF]
        obufs = scratch[2 * NBUF:3 * NBUF]
        w_v, b_v = scratch[3 * NBUF], scratch[3 * NBUF + 1]
        sis = scratch[3 * NBUF + 2: 3 * NBUF + 2 + NBUF]
        sos = scratch[3 * NBUF + 2 + NBUF:]
        _body(xT_hbm, table_hbm, w_hbm, b_hbm, out_hbm,
              idxs, gbufs, obufs, w_v, b_v, sis, sos)

    run = pl.kernel(
        body,
        out_type=jax.ShapeDtypeStruct((L, D // 8, NT, 8 * BBLK), jnp.float32),
        mesh=plsc.VectorSubcoreMesh(
            core_axis_name="c", subcore_axis_name="s",
            num_cores=NC, num_subcores=NS,
        ),
        scratch_types=(
            [pltpu.VMEM((BBLK,), jnp.int32)] * NBUF
            + [pltpu.VMEM((BBLK, D), jnp.float32)] * NBUF
            + [pltpu.VMEM((D // 8, 8 * BBLK), jnp.float32)] * NBUF
            + [pltpu.VMEM((D,), jnp.float32)] * 2
            + [pltpu.SemaphoreType.DMA] * (2 * NBUF)
        ),
        compiler_params=pltpu.CompilerParams(
            needs_layout_passes=False, use_tc_tiling_on_sc=False),
    )
    out = run(x.T, table, ln_weight, ln_bias)
    out = out.reshape(L, D // 8, NT, 8, BBLK)
    out = out.transpose(2, 4, 0, 1, 3)
    return out.reshape(B, L, D)


# speedup vs baseline: 1.7555x; 1.7555x over previous
import jax
import jax.numpy as jnp
from jax import lax
from jax.experimental import pallas as pl
from jax.experimental.pallas import tpu as pltpu, tpu_sc as plsc

NC, NS, LANES = 2, 16, 16
NW = NC * NS
D = 64
BBLK = 128
NBUF = 4
EPS = 1e-5
GRP = BBLK // LANES


def _rsqrt_v(v):
    i = plsc.bitcast(v, jnp.int32)
    y = plsc.bitcast(jnp.full((LANES,), 0x5F3759DF, jnp.int32) - (i >> 1),
                     jnp.float32)
    for _ in range(3):
        y = y * (1.5 - 0.5 * v * y * y)
    return y


def _body(xT_hbm, table_hbm, w_hbm, b_hbm, out_hbm,
          idxs, gbufs, obufs, w_v, b_v, sis, sos):
    L, B = xT_hbm.shape
    per_w = B // NW
    blocks = per_w // BBLK
    n_iter = L * blocks
    wid = lax.axis_index("s") * NC + lax.axis_index("c")
    bt0 = wid * blocks
    pltpu.sync_copy(w_hbm, w_v)
    pltpu.sync_copy(b_hbm, b_v)
    iota = lax.iota(jnp.int32, LANES)

    def start_in(i, k):
        l = i // blocks
        bt = bt0 + lax.rem(i, blocks)
        pltpu.sync_copy(xT_hbm.at[l, pl.ds(bt * BBLK, BBLK)], idxs[k])
        pltpu.async_copy(table_hbm.at[idxs[k]], gbufs[k], sis[k])

    def wait_in(k):
        pltpu.make_async_copy(table_hbm.at[pl.ds(0, BBLK)], gbufs[k],
                              sis[k]).wait()

    def start_out(i, k):
        l = i // blocks
        bt = bt0 + lax.rem(i, blocks)
        pltpu.async_copy(obufs[k], out_hbm.at[l, :, bt, :], sos[k])

    def wait_out(k):
        pltpu.make_async_copy(obufs[k], out_hbm.at[0, :, 0, :], sos[k]).wait()

    def compute(k):
        gbuf, obuf = gbufs[k], obufs[k]
        means, rstds = [], []
        zero = jnp.zeros((LANES,), jnp.float32)
        rows_g = [iota + g * LANES for g in range(GRP)]
        for g in range(GRP):
            rows = rows_g[g]

            def d_sum(d, carry):
                s, s2 = carry
                c = (rows + d) & (D - 1)
                v = plsc.load_gather(gbuf, [rows, c])
                return s + v, s2 + v * v

            s, s2 = lax.fori_loop(0, D, d_sum, (zero, zero), unroll=4)
            mean = s * (1.0 / D)
            var = s2 * (1.0 / D) - mean * mean
            means.append(mean)
            rstds.append(_rsqrt_v(var + EPS))

        def d_norm(d, _):
            for g in range(GRP):
                rows = rows_g[g]
                c = (rows + d) & (D - 1)
                wsp = plsc.load_gather(w_v, [c])
                bsp = plsc.load_gather(b_v, [c])
                v = plsc.load_gather(gbuf, [rows, c])
                o = (v - means[g]) * (rstds[g] * wsp) + bsp
                plsc.store_scatter(obuf, [c >> 3, (c & 7) * BBLK + rows], o)
            return 0

        lax.fori_loop(0, D, d_norm, 0, unroll=2)

    start_in(0, 0)
    start_in(1, 1)

    def outer(i4, _):
        for k in range(NBUF):
            i = i4 * NBUF + k
            wait_in(k)
            compute(k)
            start_out(i, k)
            kn = (k + 2) % NBUF
            if k >= 2:
                wait_out(kn)
            else:
                @pl.when(i4 > 0)
                def _():
                    wait_out(kn)

            @pl.when(i + 2 < n_iter)
            def _():
                start_in(i + 2, kn)
        return 0

    lax.fori_loop(0, n_iter // NBUF, outer, 0)
    wait_out((n_iter - 2) % NBUF)
    wait_out((n_iter - 1) % NBUF)


def kernel(x, table, ln_weight, ln_bias):
    B, L = x.shape
    NT = B // BBLK

    def body(xT_hbm, table_hbm, w_hbm, b_hbm, out_hbm, *scratch):
        idxs = scratch[0:NBUF]
        gbufs = scratch[NBUF:2 * NBUF]
        obufs = scratch[2 * NBUF:3 * NBUF]
        w_v, b_v = scratch[3 * NBUF], scratch[3 * NBUF + 1]
        sis = scratch[3 * NBUF + 2: 3 * NBUF + 2 + NBUF]
        sos = scratch[3 * NBUF + 2 + NBUF:]
        _body(xT_hbm, table_hbm, w_hbm, b_hbm, out_hbm,
              idxs, gbufs, obufs, w_v, b_v, sis, sos)

    run = pl.kernel(
        body,
        out_type=jax.ShapeDtypeStruct((L, D // 8, NT, 8 * BBLK), jnp.float32),
        mesh=plsc.VectorSubcoreMesh(
            core_axis_name="c", subcore_axis_name="s",
            num_cores=NC, num_subcores=NS,
        ),
        scratch_types=(
            [pltpu.VMEM((BBLK,), jnp.int32)] * NBUF
            + [pltpu.VMEM((BBLK, D), jnp.float32)] * NBUF
            + [pltpu.VMEM((D // 8, 8 * BBLK), jnp.float32)] * NBUF
            + [pltpu.VMEM((D,), jnp.float32)] * 2
            + [pltpu.SemaphoreType.DMA] * (2 * NBUF)
        ),
        compiler_params=pltpu.CompilerParams(
            needs_layout_passes=False, use_tc_tiling_on_sc=False),
    )
    out = run(x.T, table, ln_weight, ln_bias)
    out = out.reshape(L, D // 8, NT, 8, BBLK)
    out = out.transpose(2, 4, 0, 1, 3)
    return out.reshape(B, L, D)


# speedup vs baseline: 1.7826x; 1.0154x over previous
import jax
import jax.numpy as jnp
from jax import lax
from jax.experimental import pallas as pl
from jax.experimental.pallas import tpu as pltpu, tpu_sc as plsc

NC, NS, LANES = 2, 16, 16
NW = NC * NS
D = 64
BBLK = 128
NBUF = 4
EPS = 1e-5
GRP = BBLK // LANES


def _rsqrt_v(v):
    i = plsc.bitcast(v, jnp.int32)
    y = plsc.bitcast(jnp.full((LANES,), 0x5F3759DF, jnp.int32) - (i >> 1),
                     jnp.float32)
    for _ in range(3):
        y = y * (1.5 - 0.5 * v * y * y)
    return y


def _body(xT_hbm, table_hbm, w_hbm, b_hbm, out_hbm,
          idxs, gbufs, obufs, wb_ext, sis, sos):
    L, B = xT_hbm.shape
    per_w = B // NW
    blocks = per_w // BBLK
    n_iter = L * blocks
    wid = lax.axis_index("s") * NC + lax.axis_index("c")
    bt0 = wid * blocks
    pltpu.sync_copy(w_hbm, wb_ext.at[pl.ds(0, D)])
    pltpu.sync_copy(w_hbm.at[pl.ds(0, LANES)], wb_ext.at[pl.ds(D, LANES)])
    pltpu.sync_copy(b_hbm, wb_ext.at[pl.ds(D + LANES, D)])
    pltpu.sync_copy(b_hbm.at[pl.ds(0, LANES)],
                    wb_ext.at[pl.ds(2 * D + LANES, LANES)])
    iota = lax.iota(jnp.int32, LANES)

    def start_in(i, k):
        l = i // blocks
        bt = bt0 + lax.rem(i, blocks)
        pltpu.sync_copy(xT_hbm.at[l, pl.ds(bt * BBLK, BBLK)], idxs[k])
        pltpu.async_copy(table_hbm.at[idxs[k]], gbufs[k], sis[k])

    def wait_in(k):
        pltpu.make_async_copy(table_hbm.at[pl.ds(0, BBLK)], gbufs[k],
                              sis[k]).wait()

    def start_out(i, k):
        l = i // blocks
        bt = bt0 + lax.rem(i, blocks)
        pltpu.async_copy(obufs[k], out_hbm.at[l, :, bt, :], sos[k])

    def wait_out(k):
        pltpu.make_async_copy(obufs[k], out_hbm.at[0, :, 0, :], sos[k]).wait()

    def compute(k):
        gbuf, obuf = gbufs[k], obufs[k]
        means, rstds = [], []
        zero = jnp.zeros((LANES,), jnp.float32)
        rows_g = [iota + g * LANES for g in range(GRP)]
        NA = 4
        for g in range(GRP):
            rows = rows_g[g]

            def d_sum(d, carry):
                out = []
                for a in range(NA):
                    s, s2 = carry[2 * a], carry[2 * a + 1]
                    c = (rows + d + a * (D // NA)) & (D - 1)
                    v = plsc.load_gather(gbuf, [rows, c])
                    out += [s + v, s2 + v * v]
                return tuple(out)

            acc = lax.fori_loop(0, D // NA, d_sum, (zero,) * (2 * NA),
                                unroll=4)
            s = (acc[0] + acc[2]) + (acc[4] + acc[6])
            s2 = (acc[1] + acc[3]) + (acc[5] + acc[7])
            mean = s * (1.0 / D)
            var = s2 * (1.0 / D) - mean * mean
            means.append(mean)
            rstds.append(_rsqrt_v(var + EPS))

        def d_norm(d, _):
            for g in range(GRP):
                rows = rows_g[g]
                c = (rows + d) & (D - 1)
                off = (d + g * LANES) & (D - 1)
                wsp = wb_ext[pl.ds(off, LANES)]
                bsp = wb_ext[pl.ds(D + LANES + off, LANES)]
                v = plsc.load_gather(gbuf, [rows, c])
                o = (v - means[g]) * (rstds[g] * wsp) + bsp
                plsc.store_scatter(obuf, [c >> 3, (c & 7) * BBLK + rows], o)
            return 0

        lax.fori_loop(0, D, d_norm, 0, unroll=4)

    start_in(0, 0)
    start_in(1, 1)

    def outer(i4, _):
        for k in range(NBUF):
            i = i4 * NBUF + k
            wait_in(k)
            compute(k)
            start_out(i, k)
            kn = (k + 2) % NBUF
            if k >= 2:
                wait_out(kn)
            else:
                @pl.when(i4 > 0)
                def _():
                    wait_out(kn)

            @pl.when(i + 2 < n_iter)
            def _():
                start_in(i + 2, kn)
        return 0

    lax.fori_loop(0, n_iter // NBUF, outer, 0)
    wait_out((n_iter - 2) % NBUF)
    wait_out((n_iter - 1) % NBUF)


def kernel(x, table, ln_weight, ln_bias):
    B, L = x.shape
    NT = B // BBLK

    def body(xT_hbm, table_hbm, w_hbm, b_hbm, out_hbm, *scratch):
        idxs = scratch[0:NBUF]
        gbufs = scratch[NBUF:2 * NBUF]
        obufs = scratch[2 * NBUF:3 * NBUF]
        wb_ext = scratch[3 * NBUF]
        sis = scratch[3 * NBUF + 1: 3 * NBUF + 1 + NBUF]
        sos = scratch[3 * NBUF + 1 + NBUF:]
        _body(xT_hbm, table_hbm, w_hbm, b_hbm, out_hbm,
              idxs, gbufs, obufs, wb_ext, sis, sos)

    run = pl.kernel(
        body,
        out_type=jax.ShapeDtypeStruct((L, D // 8, NT, 8 * BBLK), jnp.float32),
        mesh=plsc.VectorSubcoreMesh(
            core_axis_name="c", subcore_axis_name="s",
            num_cores=NC, num_subcores=NS,
        ),
        scratch_types=(
            [pltpu.VMEM((BBLK,), jnp.int32)] * NBUF
            + [pltpu.VMEM((BBLK, D), jnp.float32)] * NBUF
            + [pltpu.VMEM((D // 8, 8 * BBLK), jnp.float32)] * NBUF
            + [pltpu.VMEM((2 * (D + LANES),), jnp.float32)]
            + [pltpu.SemaphoreType.DMA] * (2 * NBUF)
        ),
        compiler_params=pltpu.CompilerParams(
            needs_layout_passes=False, use_tc_tiling_on_sc=False),
    )
    out = run(x.T, table, ln_weight, ln_bias)
    out = out.reshape(L, D // 8, NT, 8, BBLK)
    out = out.transpose(2, 4, 0, 1, 3)
    return out.reshape(B, L, D)


# speedup vs baseline: 2.0302x; 1.1389x over previous
import jax
import jax.numpy as jnp
from jax import lax
from jax.experimental import pallas as pl
from jax.experimental.pallas import tpu as pltpu, tpu_sc as plsc

NC, NS, LANES = 2, 16, 16
NW = NC * NS
D = 64
BBLK = 128
NBUF = 4
EPS = 1e-5
GRP = BBLK // LANES


def _rsqrt_v(v):
    i = plsc.bitcast(v, jnp.int32)
    y = plsc.bitcast(jnp.full((LANES,), 0x5F3759DF, jnp.int32) - (i >> 1),
                     jnp.float32)
    for _ in range(3):
        y = y * (1.5 - 0.5 * v * y * y)
    return y


def _body(xT_hbm, table_hbm, w_hbm, b_hbm, out_hbm,
          idxs, gbufs, obufs, wb_ext, sis, sos):
    L, B = xT_hbm.shape
    per_w = B // NW
    blocks = per_w // BBLK
    n_iter = L * blocks
    wid = lax.axis_index("s") * NC + lax.axis_index("c")
    bt0 = wid * blocks
    pltpu.sync_copy(w_hbm, wb_ext.at[pl.ds(0, D)])
    pltpu.sync_copy(w_hbm.at[pl.ds(0, LANES)], wb_ext.at[pl.ds(D, LANES)])
    pltpu.sync_copy(b_hbm, wb_ext.at[pl.ds(D + LANES, D)])
    pltpu.sync_copy(b_hbm.at[pl.ds(0, LANES)],
                    wb_ext.at[pl.ds(2 * D + LANES, LANES)])
    iota = lax.iota(jnp.int32, LANES)

    def start_in(i, k):
        l = i // blocks
        bt = bt0 + lax.rem(i, blocks)
        pltpu.sync_copy(xT_hbm.at[l, pl.ds(bt * BBLK, BBLK)], idxs[k])
        pltpu.async_copy(table_hbm.at[idxs[k]], gbufs[k], sis[k])

    def wait_in(k):
        pltpu.make_async_copy(table_hbm.at[pl.ds(0, BBLK)], gbufs[k],
                              sis[k]).wait()

    def start_out(i, k):
        l = i // blocks
        bt = bt0 + lax.rem(i, blocks)
        pltpu.async_copy(obufs[k], out_hbm.at[l, :, bt, :], sos[k])

    def wait_out(k):
        pltpu.make_async_copy(obufs[k], out_hbm.at[0, :, 0, :], sos[k]).wait()

    def compute(k):
        gbuf, obuf = gbufs[k], obufs[k]
        means, rstds = [], []
        zero = jnp.zeros((LANES,), jnp.float32)
        rows_g = [iota + g * LANES for g in range(GRP)]
        NA = 4
        for g in range(GRP):
            rows = rows_g[g]

            def d_sum(d, carry):
                out = []
                for a in range(NA):
                    s, s2 = carry[2 * a], carry[2 * a + 1]
                    c = (rows + d + a * (D // NA)) & (D - 1)
                    v = plsc.load_gather(gbuf, [rows, c])
                    plsc.store_scatter(obuf, [c >> 3, (c & 7) * BBLK + rows],
                                       v)
                    out += [s + v, s2 + v * v]
                return tuple(out)

            acc = lax.fori_loop(0, D // NA, d_sum, (zero,) * (2 * NA),
                                unroll=2)
            s = (acc[0] + acc[2]) + (acc[4] + acc[6])
            s2 = (acc[1] + acc[3]) + (acc[5] + acc[7])
            mean = s * (1.0 / D)
            var = s2 * (1.0 / D) - mean * mean
            means.append(mean)
            rstds.append(_rsqrt_v(var + EPS))

        coefs = [rstds[g] * means[g] for g in range(GRP)]

        def d_norm(d, _):
            dv = jnp.full((LANES,), d, jnp.int32)
            wsp = plsc.load_gather(wb_ext, [dv])
            bsp = plsc.load_gather(wb_ext, [dv + (D + LANES)])
            dt = d >> 3
            col0 = (d & 7) * BBLK
            for g in range(GRP):
                sl = pl.ds(col0 + g * LANES, LANES)
                v = obuf[dt, sl]
                obuf[dt, sl] = (v * rstds[g] - coefs[g]) * wsp + bsp
            return 0

        lax.fori_loop(0, D, d_norm, 0, unroll=2)

    start_in(0, 0)
    start_in(1, 1)

    def outer(i4, _):
        for k in range(NBUF):
            i = i4 * NBUF + k
            wait_in(k)
            compute(k)
            start_out(i, k)
            kn = (k + 2) % NBUF
            if k >= 2:
                wait_out(kn)
            else:
                @pl.when(i4 > 0)
                def _():
                    wait_out(kn)

            @pl.when(i + 2 < n_iter)
            def _():
                start_in(i + 2, kn)
        return 0

    lax.fori_loop(0, n_iter // NBUF, outer, 0)
    wait_out((n_iter - 2) % NBUF)
    wait_out((n_iter - 1) % NBUF)


def kernel(x, table, ln_weight, ln_bias):
    B, L = x.shape
    NT = B // BBLK

    def body(xT_hbm, table_hbm, w_hbm, b_hbm, out_hbm, *scratch):
        idxs = scratch[0:NBUF]
        gbufs = scratch[NBUF:2 * NBUF]
        obufs = scratch[2 * NBUF:3 * NBUF]
        wb_ext = scratch[3 * NBUF]
        sis = scratch[3 * NBUF + 1: 3 * NBUF + 1 + NBUF]
        sos = scratch[3 * NBUF + 1 + NBUF:]
        _body(xT_hbm, table_hbm, w_hbm, b_hbm, out_hbm,
              idxs, gbufs, obufs, wb_ext, sis, sos)

    run = pl.kernel(
        body,
        out_type=jax.ShapeDtypeStruct((L, D // 8, NT, 8 * BBLK), jnp.float32),
        mesh=plsc.VectorSubcoreMesh(
            core_axis_name="c", subcore_axis_name="s",
            num_cores=NC, num_subcores=NS,
        ),
        scratch_types=(
            [pltpu.VMEM((BBLK,), jnp.int32)] * NBUF
            + [pltpu.VMEM((BBLK, D), jnp.float32)] * NBUF
            + [pltpu.VMEM((D // 8, 8 * BBLK), jnp.float32)] * NBUF
            + [pltpu.VMEM((2 * (D + LANES),), jnp.float32)]
            + [pltpu.SemaphoreType.DMA] * (2 * NBUF)
        ),
        compiler_params=pltpu.CompilerParams(
            needs_layout_passes=False, use_tc_tiling_on_sc=False),
    )
    out = run(x.T, table, ln_weight, ln_bias)
    out = out.reshape(L, D // 8, NT, 8, BBLK)
    out = out.transpose(2, 4, 0, 1, 3)
    return out.reshape(B, L, D)
